# baseline (speedup 1.0000x reference)
import numpy as np

import jax
import jax.numpy as jnp
from jax.experimental import pallas as pl

_B = 2
_NODES = 1024
_SEQ = 1024
_FOUR = _SEQ // 2 + 1
_FPAD = 640
_NEG = 0.2
_NFIX = 2 * _FOUR
_NPAD = 1152
_BR = 256
_NSTEPS = _B * _NODES // _BR

_T = np.arange(_SEQ, dtype=np.int64)[:, None]
_F = np.arange(_FPAD, dtype=np.int64)[None, :]
_CMAT = (np.cos(((_T * _F) % _SEQ).astype(np.float64) * (2.0 * np.pi / _SEQ))
         * (_F < _FOUR)).astype(np.float32)
_DCROW = (0.5 * _CMAT.astype(np.float64).sum(axis=0)).astype(
    np.float32).reshape(1, _FPAD)

_R = np.arange(_NPAD, dtype=np.int64)
_FR = (_R % _FOUR)[:, None]
_CFLAT = np.cos(((np.arange(_SEQ, dtype=np.int64)[None, :] * _FR) % _SEQ)
                .astype(np.float64) * (2.0 * np.pi / _SEQ)).astype(np.float32)
_NR = _R // _FOUR
_ESEL = np.stack([(_NR == 0) & (_R < _NFIX), (_NR == 1) & (_R < _NFIX)],
                 axis=1).astype(np.float32)
_ESELT = np.ascontiguousarray(_ESEL.T)


def _dg(a, b, dims):
    return jax.lax.dot_general(a, b, (dims, ((), ())),
                               preferred_element_type=jnp.float32)


def _body(occ_ref, prc_ref, occ01_ref, prc01_ref, c_ref, dc_ref, w1_ref,
          att1s_ref, att1d_ref, w2_ref, smalls_ref, wd_ref, cflat_ref,
          esel_ref, eselt_ref, adj_ref, wd2_ref, out_ref, fix_ref):
    f32 = jnp.float32
    fo = _dg(occ_ref[:, :] - 0.5, c_ref[:, :], (((1,), (0,)))) + dc_ref[:, :]
    fp = _dg(prc_ref[:, :] - 0.5, c_ref[:, :], (((1,), (0,)))) + dc_ref[:, :]
    acc = jnp.zeros_like(fo)
    for k in range(30):
        h = fo * w1_ref[0, k] + fp * w1_ref[1, k] + smalls_ref[0, k]
        g = jnp.where(h > 0, h, jnp.exp(h) - 1.0)
        acc = acc + g * w2_ref[k, 0]
    h2 = acc + smalls_ref[0, 30]
    out_ref[:, :] = _dg(h2, wd_ref[:, :], (((1,), (0,))))

    @pl.when(pl.program_id(0) == _NSTEPS - 1)
    def _fix():
        cflat = cflat_ref[:, :]
        o01 = occ01_ref[:, :]
        p01 = prc01_ref[:, :]
        xo2 = _dg(cflat, o01, (((1,), (1,))))
        xp2 = _dg(cflat, p01, (((1,), (1,))))
        xo2t = _dg(o01, cflat, (((1,), (1,))))
        xp2t = _dg(p01, cflat, (((1,), (1,))))
        es = esel_ref[:, :]
        est = eselt_ref[:, :]
        xo = jnp.sum(xo2 * es, axis=1, keepdims=True)
        xp = jnp.sum(xp2 * es, axis=1, keepdims=True)
        xot = jnp.sum(xo2t * est, axis=0, keepdims=True)
        xpt = jnp.sum(xp2t * est, axis=0, keepdims=True)
        xoxp = jnp.concatenate([xo, xp], axis=1)
        xoxpt = jnp.concatenate([xot, xpt], axis=0)
        h1 = _dg(xoxp, w1_ref[:, :], (((1,), (0,))))
        h1t = _dg(w1_ref[:, :], xoxpt, (((0,), (0,))))
        ones_n = jnp.ones((_NPAD, 1), f32)

        chunks = [(0, _NPAD // 2), (_NPAD // 2, _NPAD)]
        cnts = []
        for j0, j1 in chunks:
            w = j1 - j0
            if j1 <= _NODES:
                top = adj_ref[:, j0:j1]
            else:
                top = jnp.concatenate(
                    [adj_ref[:, j0:_NODES],
                     jnp.zeros((_NODES, j1 - _NODES), f32)], axis=1)
            blk = jnp.concatenate(
                [top, jnp.zeros((_NPAD - _NODES, w), f32)], axis=0)
            ii = jax.lax.broadcasted_iota(jnp.int32, (_NPAD, w), 0)
            jj = jax.lax.broadcasted_iota(jnp.int32, (_NPAD, w), 1) + j0
            cnts.append(blk + (ii == jj).astype(f32))

        hs = []
        for hd in range(3):
            h1h = h1[:, hd * 10:(hd + 1) * 10]
            hs.append((
                _dg(h1h, att1s_ref[hd:hd + 1, :], (((1,), (1,)))),
                _dg(att1d_ref[hd:hd + 1, :], h1t[hd * 10:(hd + 1) * 10, :],
                    (((1,), (0,)))),
                jnp.concatenate([h1h, ones_n], axis=1),
            ))
        out1_chunks, out1t_chunks = [], []
        for (j0, j1), cntb in zip(chunks, cnts):
            head_outs, head_outst = [], []
            for asrc_h, adstt_h, h1e in hs:
                m = asrc_h + adstt_h[:, j0:j1]
                l = jnp.where(m > 0, m, _NEG * m)
                lc = jnp.where(cntb > 0, l, -1e30)
                amax = jnp.max(lc, axis=0, keepdims=True)
                e = cntb * jnp.exp(lc - amax)
                agg = _dg(e, h1e, (((0,), (0,))))
                aggt = _dg(h1e, e, (((0,), (0,))))
                head_outs.append(agg[:, 0:10] / agg[:, 10:11])
                head_outst.append(aggt[0:10, :] / aggt[10:11, :])
            out1_chunks.append(jnp.concatenate(head_outs, axis=1))
            out1t_chunks.append(jnp.concatenate(head_outst, axis=0))
        b1row = smalls_ref[0:1, 0:30]
        out1 = jnp.concatenate(out1_chunks, axis=0) + b1row
        out1t = (jnp.concatenate(out1t_chunks, axis=1)
                 + _dg(b1row, jnp.ones((1, _NPAD), f32), (((0,), (0,)))))
        g = jnp.where(out1 > 0, out1, jnp.exp(out1) - 1.0)
        gt = jnp.where(out1t > 0, out1t, jnp.exp(out1t) - 1.0)
        hh = _dg(g, w2_ref[:, :], (((1,), (0,))))
        hht = _dg(w2_ref[:, :], gt, (((0,), (0,))))
        as2 = hh * smalls_ref[0, 32]
        ad2t = hht * smalls_ref[0, 33]
        hhe = jnp.concatenate([hh, ones_n], axis=1)
        out2_chunks = []
        for (j0, j1), cntb in zip(chunks, cnts):
            m = as2 + ad2t[:, j0:j1]
            l = jnp.where(m > 0, m, _NEG * m)
            lc = jnp.where(cntb > 0, l, -1e30)
            amax = jnp.max(lc, axis=0, keepdims=True)
            e = cntb * jnp.exp(lc - amax)
            agg = _dg(e, hhe, (((0,), (0,))))
            out2_chunks.append(agg[:, 0:1] / agg[:, 1:2])
        out2 = jnp.concatenate(out2_chunks, axis=0) + smalls_ref[0, 30]
        res = _dg(out2, wd2_ref[:, :], (((0,), (0,))))
        fix_ref[:, :] = res + smalls_ref[0, 31]


def kernel(occ, prc, adj, W1, att_src1, att_dst1, b1, W2, att_src2, att_dst2,
           b2, Wd, bd):
    f32 = jnp.float32
    occ2 = occ.reshape(_B * _NODES, _SEQ)
    prc2 = prc.reshape(_B * _NODES, _SEQ)
    occ01 = occ[0, 0:2, :]
    prc01 = prc[0, 0:2, :]

    wdp = jnp.zeros((_FPAD, 1), f32).at[:_FOUR, :].set(Wd)
    smalls = jnp.concatenate(
        [b1, b2, bd, att_src2[0], att_dst2[0]]).reshape(1, 34)
    wd2 = (jnp.zeros((_NPAD, 2), f32)
           .at[0:_FOUR, 0].set(Wd[:, 0])
           .at[_FOUR:_NFIX, 1].set(Wd[:, 0]))

    const = lambda i: (0, 0)
    out, fix = pl.pallas_call(
        _body,
        grid=(_NSTEPS,),
        in_specs=[
            pl.BlockSpec((_BR, _SEQ), lambda i: (i, 0)),
            pl.BlockSpec((_BR, _SEQ), lambda i: (i, 0)),
            pl.BlockSpec((2, _SEQ), const),
            pl.BlockSpec((2, _SEQ), const),
            pl.BlockSpec((_SEQ, _FPAD), const),
            pl.BlockSpec((1, _FPAD), const),
            pl.BlockSpec((2, 30), const),
            pl.BlockSpec((3, 10), const),
            pl.BlockSpec((3, 10), const),
            pl.BlockSpec((30, 1), const),
            pl.BlockSpec((1, 34), const),
            pl.BlockSpec((_FPAD, 1), const),
            pl.BlockSpec((_NPAD, _SEQ), const),
            pl.BlockSpec((_NPAD, 2), const),
            pl.BlockSpec((2, _NPAD), const),
            pl.BlockSpec((_NODES, _NODES), const),
            pl.BlockSpec((_NPAD, 2), const),
        ],
        out_specs=[pl.BlockSpec((_BR, 1), lambda i: (i, 0)),
                   pl.BlockSpec((1, 2), const)],
        out_shape=[jax.ShapeDtypeStruct((_B * _NODES, 1), f32),
                   jax.ShapeDtypeStruct((1, 2), f32)],
    )(occ2, prc2, occ01, prc01, jnp.asarray(_CMAT), jnp.asarray(_DCROW), W1,
      att_src1, att_dst1, W2, smalls, wdp, jnp.asarray(_CFLAT),
      jnp.asarray(_ESEL), jnp.asarray(_ESELT), adj, wd2)

    out = out.at[0:2, 0].set(fix[0])
    return out.reshape(_B, _NODES, 1)


# speedup vs baseline: 1.0285x; 1.0285x over previous
import numpy as np

import jax
import jax.numpy as jnp
from jax.experimental import pallas as pl

_B = 2
_NODES = 1024
_SEQ = 1024
_FOUR = _SEQ // 2 + 1
_FPAD = 640
_NEG = 0.2
_NFIX = 2 * _FOUR
_NPAD = 1152
_BR = 256
_NSTEPS = _B * _NODES // _BR

_T = np.arange(_SEQ, dtype=np.int64)[:, None]
_F = np.arange(_FPAD, dtype=np.int64)[None, :]
_CMAT = (np.cos(((_T * _F) % _SEQ).astype(np.float64) * (2.0 * np.pi / _SEQ))
         * (_F < _FOUR)).astype(np.float32)
_DCROW = (0.5 * _CMAT.astype(np.float64).sum(axis=0)).astype(
    np.float32).reshape(1, _FPAD)

_R = np.arange(_NPAD, dtype=np.int64)
_FR = (_R % _FOUR)[:, None]
_CFLAT = np.cos(((np.arange(_SEQ, dtype=np.int64)[None, :] * _FR) % _SEQ)
                .astype(np.float64) * (2.0 * np.pi / _SEQ)).astype(np.float32)
_NR = _R // _FOUR
_ESEL = np.stack([(_NR == 0) & (_R < _NFIX), (_NR == 1) & (_R < _NFIX)],
                 axis=1).astype(np.float32)
_ESELT = np.ascontiguousarray(_ESEL.T)


def _dg(a, b, dims):
    return jax.lax.dot_general(a, b, (dims, ((), ())),
                               preferred_element_type=jnp.float32)


def _body(occ_ref, prc_ref, c_ref, dc_ref, w1_ref,
          att1s_ref, att1d_ref, w2_ref, smalls_ref, wd_ref, cflat_ref,
          esel_ref, eselt_ref, adj_ref, wd2_ref, out_ref):
    f32 = jnp.float32
    fo = _dg(occ_ref[:, :] - 0.5, c_ref[:, :], (((1,), (0,)))) + dc_ref[:, :]
    fp = _dg(prc_ref[:, :] - 0.5, c_ref[:, :], (((1,), (0,)))) + dc_ref[:, :]
    acc = jnp.zeros_like(fo)
    for k in range(30):
        h = fo * w1_ref[0, k] + fp * w1_ref[1, k] + smalls_ref[0, k]
        g = jnp.where(h > 0, h, jnp.exp(h) - 1.0)
        acc = acc + g * w2_ref[k, 0]
    h2 = acc + smalls_ref[0, 30]
    out_ref[:, :] = _dg(h2, wd_ref[:, :], (((1,), (0,))))

    @pl.when(pl.program_id(0) == 0)
    def _fix():
        cflat = cflat_ref[:, :]
        o01 = occ_ref[0:2, :]
        p01 = prc_ref[0:2, :]
        xo2 = _dg(cflat, o01, (((1,), (1,))))
        xp2 = _dg(cflat, p01, (((1,), (1,))))
        xo2t = _dg(o01, cflat, (((1,), (1,))))
        xp2t = _dg(p01, cflat, (((1,), (1,))))
        es = esel_ref[:, :]
        est = eselt_ref[:, :]
        xo = jnp.sum(xo2 * es, axis=1, keepdims=True)
        xp = jnp.sum(xp2 * es, axis=1, keepdims=True)
        xot = jnp.sum(xo2t * est, axis=0, keepdims=True)
        xpt = jnp.sum(xp2t * est, axis=0, keepdims=True)
        xoxp = jnp.concatenate([xo, xp], axis=1)
        xoxpt = jnp.concatenate([xot, xpt], axis=0)
        h1 = _dg(xoxp, w1_ref[:, :], (((1,), (0,))))
        h1t = _dg(w1_ref[:, :], xoxpt, (((0,), (0,))))
        ones_n = jnp.ones((_NPAD, 1), f32)

        chunks = [(0, _NPAD // 2), (_NPAD // 2, _NPAD)]
        cnts = []
        for j0, j1 in chunks:
            w = j1 - j0
            if j1 <= _NODES:
                top = adj_ref[:, j0:j1]
            else:
                top = jnp.concatenate(
                    [adj_ref[:, j0:_NODES],
                     jnp.zeros((_NODES, j1 - _NODES), f32)], axis=1)
            blk = jnp.concatenate(
                [top, jnp.zeros((_NPAD - _NODES, w), f32)], axis=0)
            ii = jax.lax.broadcasted_iota(jnp.int32, (_NPAD, w), 0)
            jj = jax.lax.broadcasted_iota(jnp.int32, (_NPAD, w), 1) + j0
            cnts.append(blk + (ii == jj).astype(f32))

        hs = []
        for hd in range(3):
            h1h = h1[:, hd * 10:(hd + 1) * 10]
            hs.append((
                _dg(h1h, att1s_ref[hd:hd + 1, :], (((1,), (1,)))),
                _dg(att1d_ref[hd:hd + 1, :], h1t[hd * 10:(hd + 1) * 10, :],
                    (((1,), (0,)))),
                jnp.concatenate([h1h, ones_n], axis=1),
            ))
        out1_chunks, out1t_chunks = [], []
        for (j0, j1), cntb in zip(chunks, cnts):
            head_outs, head_outst = [], []
            for asrc_h, adstt_h, h1e in hs:
                m = asrc_h + adstt_h[:, j0:j1]
                l = jnp.where(m > 0, m, _NEG * m)
                lc = jnp.where(cntb > 0, l, -1e30)
                amax = jnp.max(lc, axis=0, keepdims=True)
                e = cntb * jnp.exp(lc - amax)
                agg = _dg(e, h1e, (((0,), (0,))))
                aggt = _dg(h1e, e, (((0,), (0,))))
                head_outs.append(agg[:, 0:10] / agg[:, 10:11])
                head_outst.append(aggt[0:10, :] / aggt[10:11, :])
            out1_chunks.append(jnp.concatenate(head_outs, axis=1))
            out1t_chunks.append(jnp.concatenate(head_outst, axis=0))
        b1row = smalls_ref[0:1, 0:30]
        out1 = jnp.concatenate(out1_chunks, axis=0) + b1row
        out1t = (jnp.concatenate(out1t_chunks, axis=1)
                 + _dg(b1row, jnp.ones((1, _NPAD), f32), (((0,), (0,)))))
        g = jnp.where(out1 > 0, out1, jnp.exp(out1) - 1.0)
        gt = jnp.where(out1t > 0, out1t, jnp.exp(out1t) - 1.0)
        hh = _dg(g, w2_ref[:, :], (((1,), (0,))))
        hht = _dg(w2_ref[:, :], gt, (((0,), (0,))))
        as2 = hh * smalls_ref[0, 32]
        ad2t = hht * smalls_ref[0, 33]
        hhe = jnp.concatenate([hh, ones_n], axis=1)
        out2_chunks = []
        for (j0, j1), cntb in zip(chunks, cnts):
            m = as2 + ad2t[:, j0:j1]
            l = jnp.where(m > 0, m, _NEG * m)
            lc = jnp.where(cntb > 0, l, -1e30)
            amax = jnp.max(lc, axis=0, keepdims=True)
            e = cntb * jnp.exp(lc - amax)
            agg = _dg(e, hhe, (((0,), (0,))))
            out2_chunks.append(agg[:, 0:1] / agg[:, 1:2])
        out2 = jnp.concatenate(out2_chunks, axis=0) + smalls_ref[0, 30]
        res2 = _dg(wd2_ref[:, :], out2, (((0,), (0,))))
        out_ref[0:2, :] = res2 + smalls_ref[0, 31]


def kernel(occ, prc, adj, W1, att_src1, att_dst1, b1, W2, att_src2, att_dst2,
           b2, Wd, bd):
    f32 = jnp.float32
    occ2 = occ.reshape(_B * _NODES, _SEQ)
    prc2 = prc.reshape(_B * _NODES, _SEQ)

    wdp = jnp.zeros((_FPAD, 1), f32).at[:_FOUR, :].set(Wd)
    smalls = jnp.concatenate(
        [b1, b2, bd, att_src2[0], att_dst2[0]]).reshape(1, 34)
    wd2 = (jnp.zeros((_NPAD, 2), f32)
           .at[0:_FOUR, 0].set(Wd[:, 0])
           .at[_FOUR:_NFIX, 1].set(Wd[:, 0]))

    const = lambda i: (0, 0)
    out = pl.pallas_call(
        _body,
        grid=(_NSTEPS,),
        in_specs=[
            pl.BlockSpec((_BR, _SEQ), lambda i: (i, 0)),
            pl.BlockSpec((_BR, _SEQ), lambda i: (i, 0)),
            pl.BlockSpec((_SEQ, _FPAD), const),
            pl.BlockSpec((1, _FPAD), const),
            pl.BlockSpec((2, 30), const),
            pl.BlockSpec((3, 10), const),
            pl.BlockSpec((3, 10), const),
            pl.BlockSpec((30, 1), const),
            pl.BlockSpec((1, 34), const),
            pl.BlockSpec((_FPAD, 1), const),
            pl.BlockSpec((_NPAD, _SEQ), const),
            pl.BlockSpec((_NPAD, 2), const),
            pl.BlockSpec((2, _NPAD), const),
            pl.BlockSpec((_NODES, _NODES), const),
            pl.BlockSpec((_NPAD, 2), const),
        ],
        out_specs=pl.BlockSpec((_BR, 1), lambda i: (i, 0)),
        out_shape=jax.ShapeDtypeStruct((_B * _NODES, 1), f32),
    )(occ2, prc2, jnp.asarray(_CMAT), jnp.asarray(_DCROW), W1,
      att_src1, att_dst1, W2, smalls, wdp, jnp.asarray(_CFLAT),
      jnp.asarray(_ESEL), jnp.asarray(_ESELT), adj, wd2)

    return out.reshape(_B, _NODES, 1)
